# Initial kernel scaffold; baseline (speedup 1.0000x reference)
import functools

import jax
import jax.numpy as jnp
from jax import lax
from jax.experimental import pallas as pl
from jax.experimental.pallas import tpu as pltpu
from jax.experimental.pallas import tpu_sc as plsc

N_NODES = 10000
N_EDGES = 320000
D_FEAT = 128
CHUNK = 128
LANES = 16

N_TILES = 32
CHUNKS_TOTAL = N_EDGES // CHUNK
FULL_PER_TILE = CHUNKS_TOTAL // N_TILES
REM_CHUNKS = CHUNKS_TOTAL % N_TILES
ROWS_PER_TILE = N_NODES // 16
ROW_COPY = 125


def _sc_spmm(src_hbm, dst_hbm, w_hbm, feat_hbm, out_hbm,
             sidx_v, didx_v, w_v, rows_v, acc_sh, sem):
    cid = lax.axis_index("c")
    sid = lax.axis_index("s")
    wid = sid * 2 + cid

    def zero_body(i, carry):
        for j in range(D_FEAT // LANES):
            rows_v[i, pl.ds(j * LANES, LANES)] = jnp.zeros((LANES,), jnp.float32)
        return carry
    lax.fori_loop(0, ROW_COPY, zero_body, 0)
    base_row = sid * ROWS_PER_TILE
    for t in range(ROWS_PER_TILE // ROW_COPY):
        pltpu.sync_copy(rows_v.at[pl.ds(0, ROW_COPY)],
                        acc_sh.at[pl.ds(base_row + t * ROW_COPY, ROW_COPY)])
    plsc.subcore_barrier()

    def process_chunk(base):
        pltpu.sync_copy(src_hbm.at[pl.ds(base, CHUNK)], sidx_v.at[0])
        pltpu.sync_copy(dst_hbm.at[pl.ds(base, CHUNK)], didx_v.at[0])
        pltpu.sync_copy(w_hbm.at[pl.ds(base, CHUNK)], w_v)
        pltpu.async_copy(feat_hbm.at[sidx_v.at[0]], rows_v, sem).wait()

        def row_body(i, carry):
            w = w_v[i]
            for j in range(D_FEAT // LANES):
                sl = pl.ds(j * LANES, LANES)
                rows_v[i, sl] = rows_v[i, sl] * w
            return carry
        lax.fori_loop(0, CHUNK, row_body, 0)

        pltpu.sync_copy(rows_v, acc_sh.at[didx_v.at[0]], add=True)

    def chunk_body(c, carry):
        process_chunk((c * N_TILES + wid) * CHUNK)
        return carry
    lax.fori_loop(0, FULL_PER_TILE, chunk_body, 0)

    @pl.when(wid < REM_CHUNKS)
    def _():
        process_chunk((FULL_PER_TILE * N_TILES + wid) * CHUNK)

    plsc.subcore_barrier()

    for t in range(ROWS_PER_TILE // ROW_COPY):
        r0 = base_row + t * ROW_COPY
        pltpu.sync_copy(acc_sh.at[pl.ds(r0, ROW_COPY)],
                        out_hbm.at[cid, pl.ds(r0, ROW_COPY)])


_sc_spmm_call = functools.partial(
    pl.kernel,
    out_type=jax.ShapeDtypeStruct((2, N_NODES, D_FEAT), jnp.float32),
    mesh=plsc.VectorSubcoreMesh(core_axis_name="c", subcore_axis_name="s"),
    scratch_types=[
        pltpu.VMEM((1, CHUNK), jnp.int32),
        pltpu.VMEM((1, CHUNK), jnp.int32),
        pltpu.VMEM((CHUNK,), jnp.float32),
        pltpu.VMEM((CHUNK, D_FEAT), jnp.float32),
        pltpu.VMEM_SHARED((N_NODES, D_FEAT), jnp.float32),
        pltpu.SemaphoreType.DMA,
    ],
)(_sc_spmm)


def _combine_body(p_ref, b_ref, o_ref):
    o_ref[...] = p_ref[0] + p_ref[1] + b_ref[...]


def _combine(partials, bias):
    bm = 500
    return pl.pallas_call(
        _combine_body,
        grid=(N_NODES // bm,),
        in_specs=[
            pl.BlockSpec((2, bm, D_FEAT), lambda i: (0, i, 0)),
            pl.BlockSpec((1, D_FEAT), lambda i: (0, 0)),
        ],
        out_specs=pl.BlockSpec((bm, D_FEAT), lambda i: (i, 0)),
        out_shape=jax.ShapeDtypeStruct((N_NODES, D_FEAT), jnp.float32),
    )(partials, bias.reshape(1, D_FEAT))


def kernel(feat, edge_index, weight, bias):
    src = edge_index[0].astype(jnp.int32)
    dst = edge_index[1].astype(jnp.int32)
    w = weight.reshape(-1).astype(jnp.float32)
    partials = _sc_spmm_call(src, dst, w, feat)
    return _combine(partials, bias)


# speedup vs baseline: 5.3471x; 5.3471x over previous
import functools

import jax
import jax.numpy as jnp
from jax import lax
from jax.experimental import pallas as pl
from jax.experimental.pallas import tpu as pltpu
from jax.experimental.pallas import tpu_sc as plsc

N_NODES = 10000
N_EDGES = 320000
D_FEAT = 128
CHUNK = 128
LANES = 16

N_TILES = 32
CHUNKS_TOTAL = N_EDGES // CHUNK
FULL_PER_TILE = CHUNKS_TOTAL // N_TILES
REM_CHUNKS = CHUNKS_TOTAL % N_TILES
ROWS_PER_TILE = 624
ROW_SEGS = ((0, 128), (128, 128), (256, 128), (384, 128), (512, 112))
TAIL_ROW0 = 16 * ROWS_PER_TILE


def _sc_spmm(src_hbm, dst_hbm, w_hbm, feat_hbm, out_hbm,
             sidx_v, didx_v, w_v, rows_v, acc_sh, sem):
    cid = lax.axis_index("c")
    sid = lax.axis_index("s")
    wid = sid * 2 + cid

    def zero_body(i, carry):
        for j in range(D_FEAT // LANES):
            rows_v[i, pl.ds(j * LANES, LANES)] = jnp.zeros((LANES,), jnp.float32)
        return carry
    lax.fori_loop(0, CHUNK, zero_body, 0)

    base_row = sid * ROWS_PER_TILE

    def for_each_row_slice(fn):
        for off, sz in ROW_SEGS:
            fn(base_row + off, sz)

        @pl.when(sid < 2)
        def _():
            fn(TAIL_ROW0 + sid * 8, 8)

    for_each_row_slice(
        lambda r0, sz: pltpu.sync_copy(rows_v.at[pl.ds(0, sz)],
                                       acc_sh.at[pl.ds(r0, sz)]))
    plsc.subcore_barrier()

    def process_chunk(base):
        pltpu.sync_copy(src_hbm.at[pl.ds(base, CHUNK)], sidx_v.at[0])
        pltpu.sync_copy(dst_hbm.at[pl.ds(base, CHUNK)], didx_v.at[0])
        pltpu.sync_copy(w_hbm.at[pl.ds(base, CHUNK)], w_v)
        pltpu.async_copy(feat_hbm.at[sidx_v.at[0]], rows_v, sem).wait()

        def row_body(g, carry):
            r0 = g * LANES
            wv = w_v[pl.ds(r0, LANES)]
            for k in range(LANES):
                w = wv[k]
                for j in range(D_FEAT // LANES):
                    sl = pl.ds(j * LANES, LANES)
                    rows_v[r0 + k, sl] = rows_v[r0 + k, sl] * w
            return carry
        lax.fori_loop(0, CHUNK // LANES, row_body, 0)

        pltpu.sync_copy(rows_v, acc_sh.at[didx_v.at[0]], add=True)

    def chunk_body(c, carry):
        process_chunk((c * N_TILES + wid) * CHUNK)
        return carry
    lax.fori_loop(0, FULL_PER_TILE, chunk_body, 0)

    @pl.when(wid < REM_CHUNKS)
    def _():
        process_chunk((FULL_PER_TILE * N_TILES + wid) * CHUNK)

    plsc.subcore_barrier()

    for_each_row_slice(
        lambda r0, sz: pltpu.sync_copy(acc_sh.at[pl.ds(r0, sz)],
                                       out_hbm.at[cid, pl.ds(r0, sz)]))


_sc_spmm_call = functools.partial(
    pl.kernel,
    out_type=jax.ShapeDtypeStruct((2, N_NODES, D_FEAT), jnp.float32),
    mesh=plsc.VectorSubcoreMesh(core_axis_name="c", subcore_axis_name="s"),
    scratch_types=[
        pltpu.VMEM((1, CHUNK), jnp.int32),
        pltpu.VMEM((1, CHUNK), jnp.int32),
        pltpu.VMEM((CHUNK,), jnp.float32),
        pltpu.VMEM((CHUNK, D_FEAT), jnp.float32),
        pltpu.VMEM_SHARED((N_NODES, D_FEAT), jnp.float32),
        pltpu.SemaphoreType.DMA,
    ],
)(_sc_spmm)


def _combine_body(p_ref, b_ref, o_ref):
    o_ref[...] = p_ref[0] + p_ref[1] + b_ref[...]


def _combine(partials, bias):
    bm = 1000
    return pl.pallas_call(
        _combine_body,
        grid=(N_NODES // bm,),
        in_specs=[
            pl.BlockSpec((2, bm, D_FEAT), lambda i: (0, i, 0)),
            pl.BlockSpec((1, D_FEAT), lambda i: (0, 0)),
        ],
        out_specs=pl.BlockSpec((bm, D_FEAT), lambda i: (i, 0)),
        out_shape=jax.ShapeDtypeStruct((N_NODES, D_FEAT), jnp.float32),
    )(partials, bias.reshape(1, D_FEAT))


def kernel(feat, edge_index, weight, bias):
    src = edge_index[0].astype(jnp.int32)
    dst = edge_index[1].astype(jnp.int32)
    w = weight.reshape(-1).astype(jnp.float32)
    partials = _sc_spmm_call(src, dst, w, feat)
    return _combine(partials, bias)
